# baseline (speedup 1.0000x reference)
import functools

import jax
import jax.numpy as jnp
from jax import lax
from jax.experimental import pallas as pl
from jax.experimental.pallas import tpu as pltpu
from jax.experimental.pallas import tpu_sc as plsc

NUM_CL = 100
CPA = 104
CP = 128
Q = 1024
D = 256
K = 50000
KP = 50176
KB = 1024
KC = 224
NCHUNKS = KP // KC
NKQ = 4
L = 16
LB = 32
NG = 8
CENTER = 256.0


def _dot_body(x_ref, db_ref, dt_ref, a2_ref):
    i = pl.program_id(0)
    xr = x_ref[...]
    db = db_ref[...]
    ones = jnp.ones((D, 1), jnp.float32)
    b2c = lax.dot_general(db * db, ones, (((1,), (0,)), ((), ())),
                          preferred_element_type=jnp.float32)
    dtf = lax.dot_general(db.astype(jnp.bfloat16),
                          (xr * -2.0).astype(jnp.bfloat16),
                          (((1,), (1,)), ((), ())),
                          preferred_element_type=jnp.float32)
    dt_ref[...] = dtf + (b2c - CENTER)

    @pl.when(i == 0)
    def _():
        a2_ref[...] = jnp.sum(xr * xr, axis=1)


def _segmin_body(dt_hbm, labels_hbm, out_hbm,
                 dbuf0, dbuf1, lbuf0, lbuf1, stage, sem0, sem1, *accs):
    cid = lax.axis_index("c")
    sid = lax.axis_index("s")
    kq = sid // NKQ
    cbslot = sid % NKQ
    cb = cbslot * 2 + cid
    col0 = pl.multiple_of(cb * CP, CP)

    dbufs, lbufs, sems = (dbuf0, dbuf1), (lbuf0, lbuf1), (sem0, sem1)

    def _copies(ci, ph):
        k0 = pl.multiple_of(ci * KC, KC)
        return (
            pltpu.make_async_copy(
                dt_hbm.at[pl.ds(k0, KC), pl.ds(col0, CP)], dbufs[ph], sems[ph]),
            pltpu.make_async_copy(
                labels_hbm.at[pl.ds(k0, KC)], lbufs[ph], sems[ph]),
        )

    def issue(ci, ph):
        for c in _copies(ci, ph):
            c.start()

    def drain(ci, ph):
        for c in _copies(ci, ph):
            c.wait()

    infv = jnp.full((L,), jnp.inf, dtype=jnp.float32)

    def init_body(c, _):
        for gi in range(2 * NG):
            accs[gi][pl.ds(c * L, L)] = infv
        return 0
    lax.fori_loop(0, CPA, init_body, 0)

    issue(kq, 0)
    issue(kq + NKQ, 1)

    def chunk_pair(t2, _):
        for ph in range(2):
            t = t2 * 2 + ph
            ci = kq + NKQ * t
            drain(ci, ph)
            dbuf, lbuf = dbufs[ph], lbufs[ph]

            def k_body(k16, _, dbuf=dbuf, lbuf=lbuf):
                kb = k16 * L
                lv = lbuf[pl.ds(kb, L)] * L
                for j in range(L):
                    off = lv[j]
                    for gi in range(NG):
                        ar = accs[gi + NG * (j % 2)]
                        d = dbuf[kb + j, pl.ds(gi * L, L)]
                        a = ar[pl.ds(off, L)]
                        ar[pl.ds(off, L)] = jnp.minimum(a, d)
                return 0
            lax.fori_loop(0, KC // L, k_body, 0)

            @pl.when(t < (NCHUNKS // NKQ) - 2)
            def _():
                issue(ci + 2 * NKQ, ph)
        return 0
    lax.fori_loop(0, NCHUNKS // NKQ // 2, chunk_pair, 0)

    def fin_body(c, _):
        for gi in range(NG):
            stage[c, pl.ds(gi * L, L)] = jnp.minimum(
                accs[gi][pl.ds(c * L, L)], accs[gi + NG][pl.ds(c * L, L)])
        return 0
    lax.fori_loop(0, CPA, fin_body, 0)
    pltpu.sync_copy(stage, out_hbm.at[kq, :, pl.ds(col0, CP)])


def _epi_body(part_ref, a2_ref, out_ref):
    m = part_ref[0]
    for m2 in range(1, NKQ):
        m = jnp.minimum(m, part_ref[m2])
    d2 = jnp.maximum(a2_ref[...][None, :] + (m + CENTER), 0.0)
    out_ref[...] = -jnp.sqrt(d2)


def kernel(x, embeddings_db, labels_db):
    assert x.shape == (Q, D)
    assert embeddings_db.shape == (K, D)
    assert labels_db.shape == (K,)
    labels = jnp.concatenate(
        [labels_db.astype(jnp.int32),
         jnp.full((KP - K,), NUM_CL, jnp.int32)])
    db_p = jnp.concatenate(
        [embeddings_db, jnp.zeros((KP - K, D), jnp.float32)])

    dt, a2 = pl.pallas_call(
        _dot_body,
        grid=(KP // KB,),
        in_specs=[
            pl.BlockSpec((Q, D), lambda i: (0, 0)),
            pl.BlockSpec((KB, D), lambda i: (i, 0)),
        ],
        out_specs=[
            pl.BlockSpec((KB, Q), lambda i: (i, 0)),
            pl.BlockSpec((Q,), lambda i: (0,)),
        ],
        out_shape=[
            jax.ShapeDtypeStruct((KP, Q), jnp.float32),
            jax.ShapeDtypeStruct((Q,), jnp.float32),
        ],
    )(x, db_p)

    mesh = plsc.VectorSubcoreMesh(core_axis_name="c", subcore_axis_name="s")
    segmin = functools.partial(
        pl.kernel,
        out_type=jax.ShapeDtypeStruct((NKQ, CPA, Q), jnp.float32),
        mesh=mesh,
        scratch_types=[
            pltpu.VMEM((KC, CP), jnp.float32),
            pltpu.VMEM((KC, CP), jnp.float32),
            pltpu.VMEM((KC,), jnp.int32),
            pltpu.VMEM((KC,), jnp.int32),
            pltpu.VMEM((CPA, CP), jnp.float32),
            pltpu.SemaphoreType.DMA,
            pltpu.SemaphoreType.DMA,
        ] + [pltpu.VMEM((CPA * L,), jnp.float32) for _ in range(2 * NG)],
    )(_segmin_body)

    part = segmin(dt, labels)

    QE = 128
    logits_cm = pl.pallas_call(
        _epi_body,
        grid=(Q // QE,),
        in_specs=[
            pl.BlockSpec((NKQ, CPA, QE), lambda j: (0, 0, j)),
            pl.BlockSpec((QE,), lambda j: (j,)),
        ],
        out_specs=pl.BlockSpec((CPA, QE), lambda j: (0, j)),
        out_shape=jax.ShapeDtypeStruct((CPA, Q), jnp.float32),
    )(part, a2)

    return logits_cm[:NUM_CL, :].T


# speedup vs baseline: 1.0302x; 1.0302x over previous
import functools

import jax
import jax.numpy as jnp
from jax import lax
from jax.experimental import pallas as pl
from jax.experimental.pallas import tpu as pltpu
from jax.experimental.pallas import tpu_sc as plsc

NUM_CL = 100
CPA = 104
CP = 128
Q = 1024
D = 256
K = 50000
KP = 50176
KB = 512
KC = 224
KH = KP // 2
NCHUNKS = KH // KC
NKQ = 4
L = 16
LB = 32
NG = 8
CENTER = 256.0


def _dot_body(x_ref, db_ref, dt_ref, a2_ref):
    i = pl.program_id(0)
    xr = x_ref[...]
    db = db_ref[...]
    ones = jnp.ones((D, 1), jnp.float32)
    b2c = lax.dot_general(db * db, ones, (((1,), (0,)), ((), ())),
                          preferred_element_type=jnp.float32)
    dtf = lax.dot_general(db.astype(jnp.bfloat16),
                          (xr * -2.0).astype(jnp.bfloat16),
                          (((1,), (1,)), ((), ())),
                          preferred_element_type=jnp.float32)
    dt_ref[...] = dtf + (b2c - CENTER)

    @pl.when(i == 0)
    def _():
        a2_ref[...] = jnp.sum(xr * xr, axis=1)


def _segmin_body(dt_hbm, labels_hbm, out_hbm,
                 dbuf0, dbuf1, lbuf0, lbuf1, stage, sem0, sem1, *accs):
    cid = lax.axis_index("c")
    sid = lax.axis_index("s")
    kq = sid // NKQ
    cbslot = sid % NKQ
    cb = cbslot * 2 + cid
    col0 = pl.multiple_of(cb * CP, CP)

    dbufs, lbufs, sems = (dbuf0, dbuf1), (lbuf0, lbuf1), (sem0, sem1)

    def _copies(ci, ph):
        k0 = pl.multiple_of(ci * KC, KC)
        return (
            pltpu.make_async_copy(
                dt_hbm.at[pl.ds(k0, KC), pl.ds(col0, CP)], dbufs[ph], sems[ph]),
            pltpu.make_async_copy(
                labels_hbm.at[pl.ds(k0, KC)], lbufs[ph], sems[ph]),
        )

    def issue(ci, ph):
        for c in _copies(ci, ph):
            c.start()

    def drain(ci, ph):
        for c in _copies(ci, ph):
            c.wait()

    infv = jnp.full((L,), jnp.inf, dtype=jnp.float32)

    def init_body(c, _):
        for gi in range(NG):
            accs[gi][pl.ds(c * L, L)] = infv
        return 0
    lax.fori_loop(0, CPA, init_body, 0)

    issue(kq, 0)
    issue(kq + NKQ, 1)

    def chunk_pair(t2, _):
        for ph in range(2):
            t = t2 * 2 + ph
            ci = kq + NKQ * t
            drain(ci, ph)
            dbuf, lbuf = dbufs[ph], lbufs[ph]

            def k_body(k16, _, dbuf=dbuf, lbuf=lbuf):
                kb = k16 * L
                lv = lbuf[pl.ds(kb, L)] * L
                for j in range(L):
                    off = lv[j]
                    for gi in range(NG):
                        d = dbuf[kb + j, pl.ds(gi * L, L)]
                        a = accs[gi][pl.ds(off, L)]
                        accs[gi][pl.ds(off, L)] = jnp.minimum(a, d)
                return 0
            lax.fori_loop(0, KC // L, k_body, 0)

            @pl.when(t < (NCHUNKS // NKQ) - 2)
            def _():
                issue(ci + 2 * NKQ, ph)
        return 0
    lax.fori_loop(0, NCHUNKS // NKQ // 2, chunk_pair, 0)

    def fin_body(c, _):
        for gi in range(NG):
            stage[c, pl.ds(gi * L, L)] = accs[gi][pl.ds(c * L, L)]
        return 0
    lax.fori_loop(0, CPA, fin_body, 0)
    pltpu.sync_copy(stage, out_hbm.at[kq, :, pl.ds(col0, CP)])


def _epi_body(parta_ref, partb_ref, a2_ref, out_ref):
    m = parta_ref[0]
    for m2 in range(1, NKQ):
        m = jnp.minimum(m, parta_ref[m2])
    for m2 in range(NKQ):
        m = jnp.minimum(m, partb_ref[m2])
    d2 = jnp.maximum(a2_ref[...][None, :] + (m + CENTER), 0.0)
    out_ref[...] = -jnp.sqrt(d2)


def kernel(x, embeddings_db, labels_db):
    assert x.shape == (Q, D)
    assert embeddings_db.shape == (K, D)
    assert labels_db.shape == (K,)
    labels = jnp.concatenate(
        [labels_db.astype(jnp.int32),
         jnp.full((KP - K,), NUM_CL, jnp.int32)])
    db_p = jnp.concatenate(
        [embeddings_db, jnp.zeros((KP - K, D), jnp.float32)])

    dot_half = pl.pallas_call(
        _dot_body,
        grid=(KH // KB,),
        in_specs=[
            pl.BlockSpec((Q, D), lambda i: (0, 0)),
            pl.BlockSpec((KB, D), lambda i: (i, 0)),
        ],
        out_specs=[
            pl.BlockSpec((KB, Q), lambda i: (i, 0)),
            pl.BlockSpec((Q,), lambda i: (0,)),
        ],
        out_shape=[
            jax.ShapeDtypeStruct((KH, Q), jnp.float32),
            jax.ShapeDtypeStruct((Q,), jnp.float32),
        ],
    )

    mesh = plsc.VectorSubcoreMesh(core_axis_name="c", subcore_axis_name="s")
    segmin = functools.partial(
        pl.kernel,
        out_type=jax.ShapeDtypeStruct((NKQ, CPA, Q), jnp.float32),
        mesh=mesh,
        scratch_types=[
            pltpu.VMEM((KC, CP), jnp.float32),
            pltpu.VMEM((KC, CP), jnp.float32),
            pltpu.VMEM((KC,), jnp.int32),
            pltpu.VMEM((KC,), jnp.int32),
            pltpu.VMEM((CPA, CP), jnp.float32),
            pltpu.SemaphoreType.DMA,
            pltpu.SemaphoreType.DMA,
        ] + [pltpu.VMEM((CPA * L,), jnp.float32) for _ in range(NG)],
    )(_segmin_body)

    dta, a2 = dot_half(x, db_p[:KH])
    parta = segmin(dta, labels[:KH])
    dtb, _ = dot_half(x, db_p[KH:])
    partb = segmin(dtb, labels[KH:])

    QE = 128
    logits_cm = pl.pallas_call(
        _epi_body,
        grid=(Q // QE,),
        in_specs=[
            pl.BlockSpec((NKQ, CPA, QE), lambda j: (0, 0, j)),
            pl.BlockSpec((NKQ, CPA, QE), lambda j: (0, 0, j)),
            pl.BlockSpec((QE,), lambda j: (j,)),
        ],
        out_specs=pl.BlockSpec((CPA, QE), lambda j: (0, j)),
        out_shape=jax.ShapeDtypeStruct((CPA, Q), jnp.float32),
    )(parta, partb, a2)

    return logits_cm[:NUM_CL, :].T
